# baseline (speedup 1.0000x reference)
import functools

import jax
import jax.numpy as jnp
from jax import lax
from jax.experimental import pallas as pl
from jax.experimental.pallas import tpu as pltpu
from jax.experimental.pallas import tpu_sc as plsc

BATCH = 16384
NUM_FIELDS = 26
VOCAB = 100000
EMBED_DIM = 16

NC = 2
NS = 16
LANES = 16
NW = NC * NS

R = NUM_FIELDS * EMBED_DIM
ROWS_PER_W = R // NW
BCHUNK = 4096
NBCHUNK = BATCH // BCHUNK
SLICES = BCHUNK // LANES
UNROLL = 8

SEGLEN = 33408
SEGS = [(0, SEGLEN), (SEGLEN, SEGLEN), (2 * SEGLEN, 33152)]
TAIL0, TAILLEN = 99968, 32
NSEG = len(SEGS)
NLOAD = ROWS_PER_W * NSEG


def _body(tab_hbm, idx_hbm, out_hbm, bufa, bufb, tail_v, idxf, outrow,
          rsem, tsem, wsem):
    wid = lax.axis_index("s") * NC + lax.axis_index("c")
    r0 = wid * ROWS_PER_W
    zeros16 = lax.broadcasted_iota(jnp.int32, (LANES,), 0) * 0
    bufs = (bufa, bufb)

    def seg_copy(g):
        k, s = divmod(g, NSEG)
        v0, vl = SEGS[s]
        return pltpu.async_copy(
            tab_hbm.at[pl.ds(r0 + k, 1), pl.ds(v0, vl)],
            bufs[g & 1].at[:, pl.ds(0, vl)],
            rsem,
        )

    def tail_copy(k):
        return pltpu.async_copy(
            tab_hbm.at[pl.ds(r0 + k, 1), pl.ds(TAIL0, TAILLEN)],
            tail_v,
            tsem,
        )

    def idx_load(f):
        pltpu.sync_copy(idx_hbm.at[pl.ds(f * BATCH, BATCH)], idxf)

    def drain_writes():
        for _ in range(NBCHUNK):
            pltpu.make_async_copy(
                out_hbm.at[pl.ds(0, 1), pl.ds(0, BCHUNK)],
                outrow.at[:, pl.ds(0, BCHUNK)],
                wsem,
            ).wait()

    idx_load(r0 // EMBED_DIM)
    sh = [None] * NLOAD
    sh[0] = seg_copy(0)
    sh[1] = seg_copy(1)
    th = tail_copy(0)
    for k in range(ROWS_PER_W):
        f = (r0 + k) // EMBED_DIM
        if k > 0:
            f_prev = (r0 + k - 1) // EMBED_DIM

            @pl.when(f != f_prev)
            def _():
                idx_load(f)

        for s in range(NSEG):
            g = k * NSEG + s
            v0, vl = SEGS[s]
            buf = bufs[g & 1]
            sh[g].wait()
            if s == 0 and k > 0:
                drain_writes()
            if s == NSEG - 1:
                th.wait()

            def chunk_body(c, _, s=s, v0=v0, vl=vl, buf=buf, k=k):
                @plsc.parallel_loop(0, SLICES, step=1, unroll=UNROLL)
                def _gather(j):
                    off = c * BCHUNK + j * LANES
                    vidx = idxf[pl.ds(off, LANES)]
                    if s == 0:
                        m = vidx < vl
                        loc = jnp.where(m, vidx, zeros16)
                    else:
                        m = jnp.logical_and(vidx >= v0, vidx < v0 + vl)
                        loc = jnp.where(m, vidx - v0, zeros16)
                    gat = plsc.load_gather(buf, [zeros16, loc], mask=m)
                    if s == 0:
                        res = jnp.where(m, gat, 0.0)
                    else:
                        prev = outrow[0, pl.ds(off, LANES)]
                        res = jnp.where(m, gat, prev)
                    if s == NSEG - 1:
                        mt = vidx >= TAIL0
                        loct = jnp.where(mt, vidx - TAIL0, zeros16)
                        gt = plsc.load_gather(tail_v, [zeros16, loct], mask=mt)
                        res = jnp.where(mt, gt, res)
                    outrow[0, pl.ds(off, LANES)] = res

                if s == NSEG - 1:
                    pltpu.async_copy(
                        outrow.at[:, pl.ds(c * BCHUNK, BCHUNK)],
                        out_hbm.at[pl.ds(r0 + k, 1), pl.ds(c * BCHUNK, BCHUNK)],
                        wsem,
                    )
                return 0

            lax.fori_loop(0, NBCHUNK, chunk_body, 0)
            if g + 2 < NLOAD:
                sh[g + 2] = seg_copy(g + 2)
            if s == NSEG - 1 and k + 1 < ROWS_PER_W:
                th = tail_copy(k + 1)
    drain_writes()


@functools.lru_cache(maxsize=1)
def _gather_kernel():
    return functools.partial(
        pl.kernel,
        out_type=jax.ShapeDtypeStruct((R, BATCH), jnp.float32),
        mesh=plsc.VectorSubcoreMesh(
            core_axis_name="c", subcore_axis_name="s", num_cores=NC, num_subcores=NS
        ),
        scratch_types=[
            pltpu.VMEM((1, SEGLEN), jnp.float32),
            pltpu.VMEM((1, SEGLEN), jnp.float32),
            pltpu.VMEM((1, TAILLEN), jnp.float32),
            pltpu.VMEM((BATCH,), jnp.int32),
            pltpu.VMEM((1, BATCH), jnp.float32),
            pltpu.SemaphoreType.DMA,
            pltpu.SemaphoreType.DMA,
            pltpu.SemaphoreType.DMA,
        ],
        compiler_params=pltpu.CompilerParams(
            use_tc_tiling_on_sc=True, needs_layout_passes=False
        ),
    )(_body)


def kernel(indices, tables):
    tab2d = jnp.transpose(tables, (0, 2, 1)).reshape(R, VOCAB)
    idx_lin = jnp.transpose(indices.astype(jnp.int32), (1, 0)).reshape(
        NUM_FIELDS * BATCH
    )
    out_t = _gather_kernel()(tab2d, idx_lin)
    return jnp.transpose(out_t, (1, 0))
